# Initial kernel scaffold; baseline (speedup 1.0000x reference)
import jax
import jax.numpy as jnp
from jax.experimental import pallas as pl


def kernel(nf, ef, ei, params):
    raise NotImplementedError("write your pallas kernel here")


# speedup vs baseline: 1.0000x; 1.0000x over previous
import jax
import jax.numpy as jnp
from jax.experimental import pallas as pl
from jax.experimental.pallas import tpu as pltpu

H = 64
NB = 10000
EF = 80000
ROWBLK = 2000


def _silu(x):
    return x / (1.0 + jnp.exp(-x))


def _edge_encode_body(efe_ref, efo_ref, w1_ref, b1_ref, w2_ref, b2_ref,
                      wes_ref, b1s_ref, out_ref):
    w1t = w1_ref[...].T
    w2t = w2_ref[...].T
    eme = _silu(efe_ref[...] @ w1t + b1_ref[...]) @ w2t + b2_ref[...]
    emo = _silu(efo_ref[...] @ w1t + b1_ref[...]) @ w2t + b2_ref[...]
    es = (eme + emo) * 0.5
    for l in range(4):
        out_ref[l] = es @ wes_ref[l].T + b1s_ref[l]


def _edge_encode(ef_even, ef_odd, p_ee, wes, b1s):
    nblk = EF // ROWBLK
    full = pl.BlockSpec((None,), lambda i: (0,))
    return pl.pallas_call(
        _edge_encode_body,
        grid=(nblk,),
        in_specs=[
            pl.BlockSpec((ROWBLK, 4), lambda i: (i, 0)),
            pl.BlockSpec((ROWBLK, 4), lambda i: (i, 0)),
            pl.BlockSpec((H, 4), lambda i: (0, 0)),
            pl.BlockSpec((1, H), lambda i: (0, 0)),
            pl.BlockSpec((H, H), lambda i: (0, 0)),
            pl.BlockSpec((1, H), lambda i: (0, 0)),
            pl.BlockSpec((4, H, H), lambda i: (0, 0, 0)),
            pl.BlockSpec((4, 1, H), lambda i: (0, 0, 0)),
        ],
        out_specs=pl.BlockSpec((4, ROWBLK, H), lambda i: (0, i, 0)),
        out_shape=jax.ShapeDtypeStruct((4, EF, H), jnp.float32),
    )(ef_even, ef_odd, p_ee["w1"], p_ee["b1"][None], p_ee["w2"],
      p_ee["b2"][None], wes, b1s)


def _node_encode_body(nf_ref, w1_ref, b1_ref, w2_ref, b2_ref, wa_ref, wb_ref,
                      h_ref, p_ref):
    z = _silu(nf_ref[...] @ w1_ref[...].T + b1_ref[...])
    h = z @ w2_ref[...].T + b2_ref[...]
    h_ref[...] = h
    p_ref[...] = jnp.concatenate([h @ wa_ref[...].T, h @ wb_ref[...].T], axis=1)


def _node_encode(nfr, p_ne, wa, wb):
    nblk = (2 * NB) // ROWBLK
    return pl.pallas_call(
        _node_encode_body,
        grid=(nblk,),
        in_specs=[
            pl.BlockSpec((ROWBLK, 7), lambda i: (i, 0)),
            pl.BlockSpec((H, 7), lambda i: (0, 0)),
            pl.BlockSpec((1, H), lambda i: (0, 0)),
            pl.BlockSpec((H, H), lambda i: (0, 0)),
            pl.BlockSpec((1, H), lambda i: (0, 0)),
            pl.BlockSpec((H, H), lambda i: (0, 0)),
            pl.BlockSpec((H, H), lambda i: (0, 0)),
        ],
        out_specs=[
            pl.BlockSpec((ROWBLK, H), lambda i: (i, 0)),
            pl.BlockSpec((ROWBLK, 2 * H), lambda i: (i, 0)),
        ],
        out_shape=[
            jax.ShapeDtypeStruct((2 * NB, H), jnp.float32),
            jax.ShapeDtypeStruct((2 * NB, 2 * H), jnp.float32),
        ],
    )(nfr, p_ne["w1"], p_ne["b1"][None], p_ne["w2"], p_ne["b2"][None], wa, wb)


def _node_update_body(h_ref, ag_ref, w2e_ref, wh_ref, wg_ref, b1_ref, w2_ref,
                      b2_ref, wa_ref, wb_ref, pb_ref, h_out, p_out):
    h = h_ref[...]
    agg = ag_ref[...] @ w2e_ref[...].T
    z = _silu(h @ wh_ref[...].T + agg @ wg_ref[...].T + b1_ref[...])
    hn = h + z @ w2_ref[...].T + b2_ref[...]
    h_out[...] = hn
    p_out[...] = jnp.concatenate(
        [hn @ wa_ref[...].T, hn @ wb_ref[...].T], axis=1) + pb_ref[...]


def _node_update(h, aggu, w2e, wh, wg, b1, w2, b2, wa, wb, pbias):
    nblk = (2 * NB) // ROWBLK
    wspec = pl.BlockSpec((H, H), lambda i: (0, 0))
    bspec = pl.BlockSpec((1, H), lambda i: (0, 0))
    return pl.pallas_call(
        _node_update_body,
        grid=(nblk,),
        in_specs=[
            pl.BlockSpec((ROWBLK, H), lambda i: (i, 0)),
            pl.BlockSpec((ROWBLK, H), lambda i: (i, 0)),
            wspec, wspec, wspec, bspec, wspec, bspec, wspec, wspec,
            pl.BlockSpec((1, 2 * H), lambda i: (0, 0)),
        ],
        out_specs=[
            pl.BlockSpec((ROWBLK, H), lambda i: (i, 0)),
            pl.BlockSpec((ROWBLK, 2 * H), lambda i: (i, 0)),
        ],
        out_shape=[
            jax.ShapeDtypeStruct((2 * NB, H), jnp.float32),
            jax.ShapeDtypeStruct((2 * NB, 2 * H), jnp.float32),
        ],
    )(h, aggu, w2e, wh, wg, b1[None], w2, b2[None], wa, wb, pbias[None])


def _edge_u_body(as_ref, bs_ref, ad_ref, bd_ref, e_ref, u_ref):
    e = e_ref[...]
    x1 = as_ref[0] + bd_ref[0] + e
    x2 = ad_ref[0] + bs_ref[0] + e
    u_ref[0] = _silu(x1) - _silu(x2)


def _edge_u(a_s, b_s, a_d, b_d, espl):
    nblk = EF // ROWBLK
    espec = pl.BlockSpec((1, ROWBLK, H), lambda b, i: (b, i, 0))
    return pl.pallas_call(
        _edge_u_body,
        grid=(2, nblk),
        in_specs=[espec, espec, espec, espec,
                  pl.BlockSpec((ROWBLK, H), lambda b, i: (i, 0))],
        out_specs=pl.BlockSpec((1, ROWBLK, H), lambda b, i: (b, i, 0)),
        out_shape=jax.ShapeDtypeStruct((2, EF, H), jnp.float32),
    )(a_s, b_s, a_d, b_d, espl)


def kernel(nf, ef, ei, params):
    p = params
    B, N, _ = nf.shape

    se = ei[0, 0::2]
    de = ei[1, 0::2]
    do = ei[1, 1::2]
    ef_even = ef[0::2]
    ef_odd = ef[1::2]

    was = [lp["efn"]["w1"][:, :H] for lp in p["mp"]]
    wbs = [lp["efn"]["w1"][:, H:2 * H] for lp in p["mp"]]
    wes = jnp.stack([lp["efn"]["w1"][:, 2 * H:] for lp in p["mp"]])
    b1s = jnp.stack([lp["efn"]["b1"][None] for lp in p["mp"]])

    esp = _edge_encode(ef_even, ef_odd, p["ee"], wes, b1s)

    nfr = nf.reshape(B * N, 7)
    h, pt = _node_encode(nfr, p["ne"], was[0], wbs[0])

    zeros_pb = jnp.zeros((2 * H,), jnp.float32)
    dec_wa = jnp.zeros((H, H), jnp.float32).at[:3].set(p["dec"]["w"])
    dec_pb = jnp.zeros((2 * H,), jnp.float32).at[:3].set(p["dec"]["b"])
    zeros_w = jnp.zeros((H, H), jnp.float32)

    for l, lp in enumerate(p["mp"]):
        pt3 = pt.reshape(B, N, 2 * H)
        pa, pb = pt3[..., :H], pt3[..., H:]
        u = _edge_u(pa[:, se], pb[:, se], pa[:, de], pb[:, de], esp[l])
        aggu = (jnp.zeros((B, N, H), jnp.float32)
                .at[:, de].add(u).at[:, do].add(-u))
        wh = lp["nfn"]["w1"][:, :H]
        wg = lp["nfn"]["w1"][:, H:]
        if l < 3:
            wa, wb, pb = was[l + 1], wbs[l + 1], zeros_pb
        else:
            wa, wb, pb = dec_wa, zeros_w, dec_pb
        h, pt = _node_update(h, aggu.reshape(B * N, H), lp["efn"]["w2"],
                             wh, wg, lp["nfn"]["b1"], lp["nfn"]["w2"],
                             lp["nfn"]["b2"], wa, wb, pb)

    return pt[:, :3].reshape(B, N, 3)
